# baseline (speedup 1.0000x reference)
import functools

import jax
import jax.numpy as jnp
from jax import lax
from jax.experimental import pallas as pl
from jax.experimental.pallas import tpu as pltpu
from jax.experimental.pallas import tpu_sc as plsc

N, V, H, W = 4, 50000, 512, 512
NV = N * V
F = 4 * 100000
PX = N * H * W

NC, NS, L = 2, 16, 16
NW = NC * NS

FW = NV * 3
WPT = 37504
WCH = 9376
FPAD = 409600
VIS_DUMMY = FPAD
PC = 1024
PPT = PX // NS
PB = 1024
PXT = PX // NW

_GRID_SCALE = 2.0 / 511.0

_mesh = plsc.VectorSubcoreMesh(core_axis_name="c", subcore_axis_name="s")
_params = pltpu.CompilerParams(
    needs_layout_passes=False, use_tc_tiling_on_sc=False)


@functools.partial(
    pl.kernel,
    out_type=jax.ShapeDtypeStruct((PX * 4,), jnp.float32),
    mesh=_mesh,
    compiler_params=_params,
    scratch_types=[
        pltpu.VMEM((WCH,), jnp.float32),
        pltpu.VMEM((WCH,), jnp.float32),
        pltpu.VMEM((PC,), jnp.float32),
        pltpu.VMEM((PB,), jnp.int32),
        pltpu.VMEM((PB,), jnp.int32),
        pltpu.VMEM((PB * 3,), jnp.float32),
        pltpu.VMEM((PB,), jnp.int32),
        pltpu.VMEM((PB,), jnp.int32),
        pltpu.VMEM((PB,), jnp.int32),
        pltpu.VMEM((PB,), jnp.int32),
        pltpu.VMEM((PB * 9,), jnp.int32),
        pltpu.VMEM((PB * 9,), jnp.float32),
        pltpu.VMEM((PB,), jnp.float32),
        pltpu.VMEM((PB * 4,), jnp.float32),
        pltpu.VMEM_SHARED((600016,), jnp.float32),
        pltpu.VMEM_SHARED((FPAD + 8,), jnp.float32),
        pltpu.SemaphoreType.DMA,
    ],
)
def _render(vs_hbm, vt_hbm, facesp_hbm, p2fs_hbm, p2ft_hbm, bary_hbm,
            zeros_hbm, out_hbm,
            vsb, vtb, ones1v, p2fb, idxb, baryb, iw0, iw1, w0b, w1b,
            fidx, fval, visv, outbf, flow_sh, vis_sh, sem):
    sid = lax.axis_index("s")
    wid = lax.axis_index("c") * NS + sid
    iota = lax.iota(jnp.int32, L)

    zrows = (FPAD + 8) // NS
    pltpu.sync_copy(zeros_hbm, vis_sh.at[pl.ds(sid * 25600, 25600)])

    o0 = sid * 37500 - (sid % 2) * 4

    def fl_chunk(c, _):
        o = pl.multiple_of(o0 + c * WCH, 8)
        pltpu.sync_copy(vs_hbm.at[pl.ds(o, WCH)], vsb)
        pltpu.sync_copy(vt_hbm.at[pl.ds(o, WCH)], vtb)

        def grp(g, _):
            b = g * L
            vsb[pl.ds(b, L)] = vtb[pl.ds(b, L)] - vsb[pl.ds(b, L)]
            return 0

        lax.fori_loop(0, WCH // L, grp, 0)
        pltpu.sync_copy(vsb, flow_sh.at[pl.ds(o, WCH)])
        return 0

    lax.fori_loop(0, WPT // WCH, fl_chunk, 0)

    onev = jnp.full((L,), 1.0, jnp.float32)

    def of(g, _):
        ones1v[pl.ds(g * L, L)] = onev
        return 0

    lax.fori_loop(0, PC // L, of, 0)
    plsc.subcore_barrier()

    def sc_chunk(c, _):
        off = sid * PPT + c * PC
        pltpu.sync_copy(p2fs_hbm.at[pl.ds(off, PC)], p2fb)

        def grp(g, _):
            v = p2fb[pl.ds(g * L, L)]
            idxb[pl.ds(g * L, L)] = jnp.where(v < 0, VIS_DUMMY, v)
            return 0

        lax.fori_loop(0, PC // L, grp, 0)
        pltpu.sync_copy(ones1v, vis_sh.at[idxb], add=True)
        return 0

    lax.fori_loop(0, PPT // PC, sc_chunk, 0)
    plsc.subcore_barrier()

    def px_chunk(c, _):
        p0 = wid * PXT + c * PB
        pltpu.sync_copy(p2ft_hbm.at[pl.ds(p0, PB)], p2fb)
        pltpu.sync_copy(bary_hbm.at[pl.ds(p0 * 3, PB * 3)], baryb)

        def ig(g, _):
            b = g * L
            v = jnp.maximum(p2fb[pl.ds(b, L)], 0)
            idxb[pl.ds(b, L)] = v
            iw0[pl.ds(b, L)] = v * 2
            iw1[pl.ds(b, L)] = v * 2 + 1
            return 0

        lax.fori_loop(0, PB // L, ig, 0)
        c0 = pltpu.async_copy(facesp_hbm.at[iw0], w0b, sem)
        c1 = pltpu.async_copy(facesp_hbm.at[iw1], w1b, sem)
        c0.wait()
        c1.wait()

        def up(g, _):
            b = g * L
            w0 = w0b[pl.ds(b, L)]
            w1 = w1b[pl.ds(b, L)]
            v0 = w0 & 0x3FFFF
            v1 = (lax.shift_right_logical(w0, 18) & 0x3FFF) | (
                (w1 & 0xF) << 14)
            v2 = lax.shift_right_logical(w1, 4)
            for j, vv in enumerate((v0, v1, v2)):
                e = vv * 3
                fidx[pl.ds(j * 3 * PB + b, L)] = e
                fidx[pl.ds((j * 3 + 1) * PB + b, L)] = e + 1
                fidx[pl.ds((j * 3 + 2) * PB + b, L)] = e + 2
            return 0

        lax.fori_loop(0, PB // L, up, 0)

        cps = [
            pltpu.async_copy(
                flow_sh.at[fidx.at[pl.ds(k * PB, PB)]],
                fval.at[pl.ds(k * PB, PB)], sem)
            for k in range(9)
        ]
        cv = pltpu.async_copy(vis_sh.at[idxb], visv, sem)
        for cp in cps:
            cp.wait()
        cv.wait()

        def grp(g, _):
            b = g * L
            r = b + iota
            b0 = plsc.load_gather(baryb, [r * 3])
            b1 = plsc.load_gather(baryb, [r * 3 + 1])
            b2 = plsc.load_gather(baryb, [r * 3 + 2])
            ox = (b0 * fval[pl.ds(0 * PB + b, L)]
                  + b1 * fval[pl.ds(3 * PB + b, L)]
                  + b2 * fval[pl.ds(6 * PB + b, L)])
            oy = (b0 * fval[pl.ds(1 * PB + b, L)]
                  + b1 * fval[pl.ds(4 * PB + b, L)]
                  + b2 * fval[pl.ds(7 * PB + b, L)])
            oz = (b0 * fval[pl.ds(2 * PB + b, L)]
                  + b1 * fval[pl.ds(5 * PB + b, L)]
                  + b2 * fval[pl.ds(8 * PB + b, L)])
            vcnt = visv[pl.ds(b, L)]
            pm = p2fb[pl.ds(b, L)] >= 0
            q = p0 + r
            gx = (q & 511).astype(jnp.float32) * _GRID_SCALE - 1.0
            gy = ((q >> 9) & 511).astype(jnp.float32) * _GRID_SCALE - 1.0
            zero = jnp.zeros((L,), jnp.float32)
            one = jnp.full((L,), 1.0, jnp.float32)
            ox = jnp.where(pm, ox, zero) + gx
            oy = jnp.where(pm, oy, zero) + gy
            oz = jnp.where(pm, oz, zero)
            ov = jnp.where(pm & (vcnt > 0.5), one, zero)
            r4 = r * 4
            for cc, vec in ((0, ox), (1, oy), (2, oz), (3, ov)):
                plsc.store_scatter(outbf, [r4 + cc], vec)
            return 0

        lax.fori_loop(0, PB // L, grp, 0)
        pltpu.sync_copy(outbf, out_hbm.at[pl.ds(p0 * 4, PB * 4)])
        return 0

    lax.fori_loop(0, PXT // PB, px_chunk, 0)


def kernel(verts_source_ndc, verts_target_ndc, faces, pix_to_face_source,
           pix_to_face_target, bary_coords):
    vs = verts_source_ndc.reshape(-1)
    vt = verts_target_ndc.reshape(-1)
    v0 = faces[:, 0]
    v1 = faces[:, 1]
    v2 = faces[:, 2]
    w0 = v0 | ((v1 & 0x3FFF) << 18)
    w1 = jax.lax.shift_right_logical(v1, 14) | (v2 << 4)
    facesp = jnp.stack([w0, w1], axis=1).reshape(-1)
    p2fs = pix_to_face_source.reshape(-1)
    p2ft = pix_to_face_target.reshape(-1)
    bary = bary_coords.reshape(-1)
    zeros = jnp.zeros((25600,), jnp.float32)

    out = _render(vs, vt, facesp, p2fs, p2ft, bary, zeros)
    return out.reshape(N, H, W, 4)
